# baseline (speedup 1.0000x reference)
import functools

import ml_dtypes
import numpy as np

import jax
import jax.numpy as jnp
from jax import lax
from jax.experimental import pallas as pl
from jax.experimental.pallas import tpu as pltpu
from jax.experimental.pallas import tpu_sc as plsc

VOCAB = 100000
D = 768
B = 4
S = 2048
N = B * S
NC = 2
NS = 16
NW = NC * NS
TPW = N // NW
CH = 16
NCH = TPW // CH
NB = 2
LANES = 16
VEC = D // LANES


@functools.lru_cache(maxsize=1)
def _positional_encoding(max_seq, d_model):
    pos = np.arange(max_seq, dtype=np.float32)[:, None]
    i = np.arange(0, d_model, 2, dtype=np.float32)[None, :]
    denom = np.power(np.float32(10000.0), i / np.float32(d_model))
    arg = (pos / denom).astype(np.float32)
    pe = np.stack([np.sin(arg), np.cos(arg)], axis=2).astype(np.float32)
    x = pe.reshape(-1, 2, LANES)
    shuf = np.stack([x[:, 0, :], x[:, 1, :]], axis=-1).reshape(-1)
    bf = shuf.astype(ml_dtypes.bfloat16)
    return jnp.asarray(bf.view(np.int32))


def _body(table, tokens, pe, out, idx_v,
          rows0, rows1, pe_t,
          sg0, sg1, so0, so1, spe):
    rows = (rows0, rows1)
    sgs = (sg0, sg1)
    sos = (so0, so1)
    sid = lax.axis_index("s")
    wid = sid * NC + lax.axis_index("c")
    base = wid * TPW
    s0 = lax.rem(base, S)

    pe_load = pltpu.async_copy(pe.at[pl.ds(s0 * (D // 2), TPW * (D // 2))],
                               pe_t, spe)
    pltpu.sync_copy(tokens.at[pl.ds(base, TPW)], idx_v)

    def start_gather(c):
        pltpu.async_copy(table.at[idx_v.at[pl.ds(c * CH, CH)]],
                         rows[c % NB], sgs[c % NB])

    start_gather(0)
    start_gather(1)
    pe_load.wait()

    def chunk_step(c, b):
        cb = c * CH
        pltpu.make_async_copy(table.at[idx_v.at[pl.ds(cb, CH)]],
                              rows[b], sgs[b]).wait()
        rv = rows[b]

        @plsc.parallel_loop(0, CH, step=1, unroll=8)
        def _add(t):
            rbase = (cb + t) * (D // 2)
            for j in range(VEC // 2):
                pv32 = pe_t[pl.ds(rbase + j * LANES, LANES)]
                a0 = plsc.bitcast(pv32 << 16, jnp.float32)
                a1 = plsc.bitcast(pv32 & jnp.int32(-65536), jnp.float32)
                sl0 = (t, pl.ds((2 * j) * LANES, LANES))
                sl1 = (t, pl.ds((2 * j + 1) * LANES, LANES))
                rv[sl0] = rv[sl0] + a0
                rv[sl1] = rv[sl1] + a1

        pltpu.async_copy(rv, out.at[pl.ds(base + cb, CH)], sos[b])
        n = c + 2
        bn = (b + 2) % NB

        @pl.when(n < NCH)
        def _prefetch():
            @pl.when(c >= 2)
            def _drain():
                pltpu.make_async_copy(
                    rows[bn], out.at[pl.ds(base + (c - 2) * CH, CH)],
                    sos[bn]).wait()
            pltpu.async_copy(table.at[idx_v.at[pl.ds(n * CH, CH)]],
                             rows[bn], sgs[bn])

    def group(g, carry):
        for b in range(NB):
            chunk_step(g * NB + b, b)
        return carry

    lax.fori_loop(0, NCH // NB, group, 0)
    for k in range(NB):
        c = NCH - NB + k
        pltpu.make_async_copy(rows[c % NB], out.at[pl.ds(base + c * CH, CH)],
                              sos[c % NB]).wait()


@jax.jit
def kernel(tokens, table):
    pe = _positional_encoding(S, D)
    tok = tokens.reshape(N).astype(jnp.int32)
    mesh = plsc.VectorSubcoreMesh(core_axis_name="c", subcore_axis_name="s")
    f = pl.kernel(
        _body,
        out_type=jax.ShapeDtypeStruct((N, D), jnp.float32),
        mesh=mesh,
        compiler_params=pltpu.CompilerParams(needs_layout_passes=False),
        scratch_types=[
            pltpu.VMEM((TPW,), jnp.int32),
            pltpu.VMEM((CH, D), jnp.float32),
            pltpu.VMEM((CH, D), jnp.float32),
            pltpu.VMEM((TPW * D // 2,), jnp.int32),
            pltpu.SemaphoreType.DMA,
            pltpu.SemaphoreType.DMA,
            pltpu.SemaphoreType.DMA,
            pltpu.SemaphoreType.DMA,
            pltpu.SemaphoreType.DMA,
        ],
    )
    out = f(table, tok, pe)
    return out.reshape(B, S, D)


# speedup vs baseline: 1.1541x; 1.1541x over previous
import functools

import ml_dtypes
import numpy as np

import jax
import jax.numpy as jnp
from jax import lax
from jax.experimental import pallas as pl
from jax.experimental.pallas import tpu as pltpu
from jax.experimental.pallas import tpu_sc as plsc

VOCAB = 100000
D = 768
B = 4
S = 2048
N = B * S
NC = 2
NS = 16
NW = NC * NS
TPW = N // NW
CH = 16
NCH = TPW // CH
NB = 2
LANES = 16
VEC = D // LANES


@functools.lru_cache(maxsize=1)
def _positional_encoding(max_seq, d_model):
    pos = np.arange(max_seq, dtype=np.float32)[:, None]
    i = np.arange(0, d_model, 2, dtype=np.float32)[None, :]
    denom = np.power(np.float32(10000.0), i / np.float32(d_model))
    arg = (pos / denom).astype(np.float32)
    pe = np.stack([np.sin(arg), np.cos(arg)], axis=2).astype(np.float32)
    x = pe.reshape(-1, 2, LANES)
    shuf = np.stack([x[:, 0, :], x[:, 1, :]], axis=-1).reshape(-1)
    bf = shuf.astype(ml_dtypes.bfloat16)
    return jnp.asarray(bf.view(np.int32))


def _body(table, tokens, pe, out, idx_v,
          rows0, rows1, pe_t,
          sg0, sg1, so0, so1, spe):
    rows = (rows0, rows1)
    sgs = (sg0, sg1)
    sos = (so0, so1)
    sid = lax.axis_index("s")
    wid = sid * NC + lax.axis_index("c")
    base = wid * TPW
    s0 = lax.rem(base, S)

    pe_load = pltpu.async_copy(pe.at[pl.ds(s0 * (D // 2), TPW * (D // 2))],
                               pe_t, spe)
    pltpu.sync_copy(tokens.at[pl.ds(base, TPW)], idx_v)

    def start_gather(c):
        pltpu.async_copy(table.at[idx_v.at[pl.ds(c * CH, CH)]],
                         rows[c % NB], sgs[c % NB])

    start_gather(0)
    start_gather(1)
    pe_load.wait()

    def chunk_step(c, b):
        cb = c * CH
        pltpu.make_async_copy(table.at[idx_v.at[pl.ds(cb, CH)]],
                              rows[b], sgs[b]).wait()
        rv = rows[b]

        @plsc.parallel_loop(0, CH, step=1, unroll=4)
        def _add(t):
            rbase = (cb + t) * (D // 2)
            for j in range(VEC // 2):
                pv32 = pe_t[pl.ds(rbase + j * LANES, LANES)]
                a0 = plsc.bitcast(pv32 << 16, jnp.float32)
                a1 = plsc.bitcast(pv32 & jnp.int32(-65536), jnp.float32)
                sl0 = (t, pl.ds((2 * j) * LANES, LANES))
                sl1 = (t, pl.ds((2 * j + 1) * LANES, LANES))
                rv[sl0] = rv[sl0] + a0
                rv[sl1] = rv[sl1] + a1

        pltpu.async_copy(rv, out.at[pl.ds(base + cb, CH)], sos[b])
        n = c + 2
        bn = (b + 2) % NB

        @pl.when(n < NCH)
        def _prefetch():
            @pl.when(c >= 2)
            def _drain():
                pltpu.make_async_copy(
                    rows[bn], out.at[pl.ds(base + (c - 2) * CH, CH)],
                    sos[bn]).wait()
            pltpu.async_copy(table.at[idx_v.at[pl.ds(n * CH, CH)]],
                             rows[bn], sgs[bn])

    def group(g, carry):
        for b in range(NB):
            chunk_step(g * NB + b, b)
        return carry

    lax.fori_loop(0, NCH // NB, group, 0)
    for k in range(NB):
        c = NCH - NB + k
        pltpu.make_async_copy(rows[c % NB], out.at[pl.ds(base + c * CH, CH)],
                              sos[c % NB]).wait()


@jax.jit
def kernel(tokens, table):
    pe = _positional_encoding(S, D)
    tok = tokens.reshape(N).astype(jnp.int32)
    mesh = plsc.VectorSubcoreMesh(core_axis_name="c", subcore_axis_name="s")
    f = pl.kernel(
        _body,
        out_type=jax.ShapeDtypeStruct((N, D), jnp.float32),
        mesh=mesh,
        compiler_params=pltpu.CompilerParams(needs_layout_passes=False),
        scratch_types=[
            pltpu.VMEM((TPW,), jnp.int32),
            pltpu.VMEM((CH, D), jnp.float32),
            pltpu.VMEM((CH, D), jnp.float32),
            pltpu.VMEM((TPW * D // 2,), jnp.int32),
            pltpu.SemaphoreType.DMA,
            pltpu.SemaphoreType.DMA,
            pltpu.SemaphoreType.DMA,
            pltpu.SemaphoreType.DMA,
            pltpu.SemaphoreType.DMA,
        ],
    )
    out = f(table, tok, pe)
    return out.reshape(B, S, D)
